# baseline (speedup 1.0000x reference)
import jax
import jax.numpy as jnp
from jax.experimental import pallas as pl
from jax.experimental.pallas import tpu as pltpu

B = 8192
IN = 128
D = 512
NRES = 4
M = 256
K = 1024
EDIM = 64
BETA = 0.25
TILE = 2048
NTILES = B // TILE
NGROUPS = M // EDIM
NZ = B * NGROUPS


def _dot(a, b):
    return jnp.dot(a.astype(jnp.bfloat16), b,
                   preferred_element_type=jnp.float32)


def _ln(x, g, b):
    mu = jnp.mean(x, axis=-1, keepdims=True)
    var = jnp.mean((x - mu) ** 2, axis=-1, keepdims=True)
    return (x - mu) / jnp.sqrt(var + 1e-5) * g + b


def _fused_body(x_ref, ew_w, ew_b, er_w, er_b, el_g, el_b, eo_w, eo_b,
                dw_w, dw_b, dr_w, dr_b, dl_g, dl_b, do_w, do_b,
                cb_ref, cbt2_ref, cn_ref,
                yhat_ref, xhat_ref, loss_ref, perp_ref,
                ew16, er16, eo16, dw16, dr16, do16, cb16, cbt16,
                counts_acc, sq_acc):
    i = pl.program_id(0)

    @pl.when(i == 0)
    def _prep():
        bf = jnp.bfloat16
        ew16[...] = ew_w[...].astype(bf)
        er16[...] = er_w[...].astype(bf)
        eo16[...] = eo_w[...].astype(bf)
        dw16[...] = dw_w[...].astype(bf)
        dr16[...] = dr_w[...].astype(bf)
        do16[...] = do_w[...].astype(bf)
        cb16[...] = cb_ref[...].astype(bf)
        cbt16[...] = cbt2_ref[...].astype(bf)
        counts_acc[...] = jnp.zeros((8, K), jnp.float32)
        sq_acc[...] = jnp.zeros((8, 128), jnp.float32)

    x = x_ref[...]

    h = _dot(x, ew16[...]) + ew_b[...]
    for j in range(NRES):
        t = _dot(h, er16[j]) + er_b[j:j + 1, :]
        r = jnp.maximum(_ln(t, el_g[j:j + 1, :], el_b[j:j + 1, :]), 0.0)
        h = h + r
    y = _dot(h, eo16[...]) + eo_b[...]

    cn = cn_ref[...]
    iota = jax.lax.broadcasted_iota(jnp.int32, (TILE, K), 1)
    zq_groups = []
    counts = jnp.zeros((1, K), dtype=jnp.float32)
    sq = jnp.float32(0.0)
    for g in range(NGROUPS):
        z = y[:, g * EDIM:(g + 1) * EDIM]
        z2 = jnp.sum(z ** 2, axis=1, keepdims=True)
        s2 = _dot(z, cbt16[...])
        d = z2 + cn - s2
        dmin = jnp.min(d, axis=1, keepdims=True)
        idx = jnp.min(jnp.where(d == dmin, iota, K), axis=1, keepdims=True)
        oh = (iota == idx).astype(jnp.float32)
        zq = jnp.dot(oh.astype(jnp.bfloat16), cb16[...],
                     preferred_element_type=jnp.float32)
        zq_groups.append(zq)
        counts = counts + jnp.sum(oh, axis=0, keepdims=True)
        sq = sq + jnp.sum((zq - z) ** 2)

    yh = jnp.concatenate(zq_groups, axis=1)
    yhat_ref[...] = yh
    counts_acc[...] += jnp.broadcast_to(counts, (8, K))
    sq_acc[...] += jnp.broadcast_to(sq.reshape(1, 1), (8, 128))

    h = _dot(yh, dw16[...]) + dw_b[...]
    for j in range(NRES):
        t = _dot(h, dr16[j]) + dr_b[j:j + 1, :]
        mu = jnp.mean(t, axis=-1, keepdims=True)
        var = jnp.mean(t * t, axis=-1, keepdims=True) - mu * mu
        r = jnp.maximum(
            (t - mu) * jax.lax.rsqrt(var + 1e-5) * dl_g[j:j + 1, :]
            + dl_b[j:j + 1, :], 0.0)
        h = h + r
    xhat_ref[...] = _dot(h, do16[...]) + do_b[...]

    @pl.when(i == NTILES - 1)
    def _finalize():
        e_mean = counts_acc[0:1, :] / jnp.float32(NZ)
        perp = jnp.exp(-jnp.sum(e_mean * jnp.log(e_mean + 1e-10)))
        m = sq_acc[0, 0] / jnp.float32(NZ * EDIM)
        loss = m + BETA * m
        loss_ref[...] = jnp.broadcast_to(loss.reshape(1, 1), (8, 128))
        perp_ref[...] = jnp.broadcast_to(perp.reshape(1, 1), (8, 128))


def _full(shape):
    nd = len(shape)
    return pl.BlockSpec(shape, lambda i: (0,) * nd)


@jax.jit
def kernel(weight_block, enc_win_w, enc_win_b, enc_res_w, enc_res_b,
           enc_ln_g, enc_ln_b, enc_out_w, enc_out_b,
           dec_win_w, dec_win_b, dec_res_w, dec_res_b,
           dec_ln_g, dec_ln_b, dec_out_w, dec_out_b, codebook):
    cbt2 = (codebook + codebook).T
    cn = jnp.sum(codebook ** 2, axis=1).reshape(1, K)
    operands = (
        weight_block,
        enc_win_w, enc_win_b.reshape(1, D), enc_res_w, enc_res_b,
        enc_ln_g, enc_ln_b, enc_out_w, enc_out_b.reshape(1, M),
        dec_win_w, dec_win_b.reshape(1, D), dec_res_w, dec_res_b,
        dec_ln_g, dec_ln_b, dec_out_w, dec_out_b.reshape(1, IN),
        codebook, cbt2, cn,
    )
    in_specs = [
        pl.BlockSpec((TILE, IN), lambda i: (i, 0)),
        _full((IN, D)), _full((1, D)), _full((NRES, D, D)), _full((NRES, D)),
        _full((NRES, D)), _full((NRES, D)), _full((D, M)), _full((1, M)),
        _full((M, D)), _full((1, D)), _full((NRES, D, D)), _full((NRES, D)),
        _full((NRES, D)), _full((NRES, D)), _full((D, IN)), _full((1, IN)),
        _full((K, EDIM)), _full((EDIM, K)), _full((1, K)),
    ]
    out_shapes = (
        jax.ShapeDtypeStruct((B, M), jnp.float32),
        jax.ShapeDtypeStruct((B, IN), jnp.float32),
        jax.ShapeDtypeStruct((8, 128), jnp.float32),
        jax.ShapeDtypeStruct((8, 128), jnp.float32),
    )
    out_specs = (
        pl.BlockSpec((TILE, M), lambda i: (i, 0)),
        pl.BlockSpec((TILE, IN), lambda i: (i, 0)),
        pl.BlockSpec((8, 128), lambda i: (0, 0)),
        pl.BlockSpec((8, 128), lambda i: (0, 0)),
    )
    bf = jnp.bfloat16
    scratch_shapes = [
        pltpu.VMEM((IN, D), bf), pltpu.VMEM((NRES, D, D), bf),
        pltpu.VMEM((D, M), bf), pltpu.VMEM((M, D), bf),
        pltpu.VMEM((NRES, D, D), bf), pltpu.VMEM((D, IN), bf),
        pltpu.VMEM((K, EDIM), bf), pltpu.VMEM((EDIM, K), bf),
        pltpu.VMEM((8, K), jnp.float32), pltpu.VMEM((8, 128), jnp.float32),
    ]
    y_hat, x_hat, loss_b, perp_b = pl.pallas_call(
        _fused_body,
        grid=(NTILES,),
        in_specs=in_specs,
        out_specs=out_specs,
        out_shape=out_shapes,
        scratch_shapes=scratch_shapes,
    )(*operands)

    return (loss_b[0, 0], x_hat, perp_b[0, 0], y_hat)


# speedup vs baseline: 1.3568x; 1.3568x over previous
import jax
import jax.numpy as jnp
from jax.experimental import pallas as pl
from jax.experimental.pallas import tpu as pltpu

B = 8192
IN = 128
D = 512
NRES = 4
M = 256
K = 1024
EDIM = 64
BETA = 0.25
TILE = 1024
NTILES = B // TILE
NGROUPS = M // EDIM
NZ = B * NGROUPS


def _dot(a, b):
    return jnp.dot(a.astype(jnp.bfloat16), b,
                   preferred_element_type=jnp.float32)


def _ln(x, g, b):
    mu = jnp.mean(x, axis=-1, keepdims=True)
    var = jnp.mean((x - mu) ** 2, axis=-1, keepdims=True)
    return (x - mu) / jnp.sqrt(var + 1e-5) * g + b


def _fused_body(x_ref, ew_w, ew_b, er_w, er_b, el_g, el_b, eo_w, eo_b,
                dw_w, dw_b, dr_w, dr_b, dl_g, dl_b, do_w, do_b,
                cb_ref, cbt2_ref, cn_ref,
                yhat_ref, xhat_ref, loss_ref, perp_ref,
                ew16, er16, eo16, dw16, dr16, do16, cb16, cbt16,
                counts_acc, sq_acc):
    i = pl.program_id(0)

    @pl.when(i == 0)
    def _prep():
        bf = jnp.bfloat16
        ew16[...] = ew_w[...].astype(bf)
        er16[...] = er_w[...].astype(bf)
        eo16[...] = eo_w[...].astype(bf)
        dw16[...] = dw_w[...].astype(bf)
        dr16[...] = dr_w[...].astype(bf)
        do16[...] = do_w[...].astype(bf)
        cb16[...] = cb_ref[...].astype(bf)
        cbt16[...] = cbt2_ref[...].astype(bf)
        counts_acc[...] = jnp.zeros((8, K), jnp.float32)
        sq_acc[...] = jnp.zeros((8, 128), jnp.float32)

    x = x_ref[...]

    h = _dot(x, ew16[...]) + ew_b[...]
    for j in range(NRES):
        t = _dot(h, er16[j]) + er_b[j:j + 1, :]
        r = jnp.maximum(_ln(t, el_g[j:j + 1, :], el_b[j:j + 1, :]), 0.0)
        h = h + r
    y = _dot(h, eo16[...]) + eo_b[...]

    cn = cn_ref[...]
    iota = jax.lax.broadcasted_iota(jnp.int32, (TILE, K), 1)
    zq_groups = []
    counts = jnp.zeros((1, K), dtype=jnp.float32)
    sq = jnp.float32(0.0)
    for g in range(NGROUPS):
        z = y[:, g * EDIM:(g + 1) * EDIM]
        z2 = jnp.sum(z ** 2, axis=1, keepdims=True)
        s2 = _dot(z, cbt16[...])
        d = z2 + cn - s2
        dmin = jnp.min(d, axis=1, keepdims=True)
        idx = jnp.min(jnp.where(d == dmin, iota, K), axis=1, keepdims=True)
        oh = (iota == idx).astype(jnp.float32)
        zq = jnp.dot(oh.astype(jnp.bfloat16), cb16[...],
                     preferred_element_type=jnp.float32)
        zq_groups.append(zq)
        counts = counts + jnp.sum(oh, axis=0, keepdims=True)
        sq = sq + jnp.sum((zq - z) ** 2)

    yh = jnp.concatenate(zq_groups, axis=1)
    yhat_ref[...] = yh
    counts_acc[...] += jnp.broadcast_to(counts, (8, K))
    sq_acc[...] += jnp.broadcast_to(sq.reshape(1, 1), (8, 128))

    h = _dot(yh, dw16[...]) + dw_b[...]
    for j in range(NRES):
        t = _dot(h, dr16[j]) + dr_b[j:j + 1, :]
        mu = jnp.mean(t, axis=-1, keepdims=True)
        var = jnp.mean(t * t, axis=-1, keepdims=True) - mu * mu
        r = jnp.maximum(
            (t - mu) * jax.lax.rsqrt(var + 1e-5) * dl_g[j:j + 1, :]
            + dl_b[j:j + 1, :], 0.0)
        h = h + r
    xhat_ref[...] = _dot(h, do16[...]) + do_b[...]

    @pl.when(i == NTILES - 1)
    def _finalize():
        e_mean = counts_acc[0:1, :] / jnp.float32(NZ)
        perp = jnp.exp(-jnp.sum(e_mean * jnp.log(e_mean + 1e-10)))
        m = sq_acc[0, 0] / jnp.float32(NZ * EDIM)
        loss = m + BETA * m
        loss_ref[...] = jnp.broadcast_to(loss.reshape(1, 1), (8, 128))
        perp_ref[...] = jnp.broadcast_to(perp.reshape(1, 1), (8, 128))


def _full(shape):
    nd = len(shape)
    return pl.BlockSpec(shape, lambda i: (0,) * nd)


@jax.jit
def kernel(weight_block, enc_win_w, enc_win_b, enc_res_w, enc_res_b,
           enc_ln_g, enc_ln_b, enc_out_w, enc_out_b,
           dec_win_w, dec_win_b, dec_res_w, dec_res_b,
           dec_ln_g, dec_ln_b, dec_out_w, dec_out_b, codebook):
    cbt2 = (codebook + codebook).T
    cn = jnp.sum(codebook ** 2, axis=1).reshape(1, K)
    operands = (
        weight_block,
        enc_win_w, enc_win_b.reshape(1, D), enc_res_w, enc_res_b,
        enc_ln_g, enc_ln_b, enc_out_w, enc_out_b.reshape(1, M),
        dec_win_w, dec_win_b.reshape(1, D), dec_res_w, dec_res_b,
        dec_ln_g, dec_ln_b, dec_out_w, dec_out_b.reshape(1, IN),
        codebook, cbt2, cn,
    )
    in_specs = [
        pl.BlockSpec((TILE, IN), lambda i: (i, 0)),
        _full((IN, D)), _full((1, D)), _full((NRES, D, D)), _full((NRES, D)),
        _full((NRES, D)), _full((NRES, D)), _full((D, M)), _full((1, M)),
        _full((M, D)), _full((1, D)), _full((NRES, D, D)), _full((NRES, D)),
        _full((NRES, D)), _full((NRES, D)), _full((D, IN)), _full((1, IN)),
        _full((K, EDIM)), _full((EDIM, K)), _full((1, K)),
    ]
    out_shapes = (
        jax.ShapeDtypeStruct((B, M), jnp.float32),
        jax.ShapeDtypeStruct((B, IN), jnp.float32),
        jax.ShapeDtypeStruct((8, 128), jnp.float32),
        jax.ShapeDtypeStruct((8, 128), jnp.float32),
    )
    out_specs = (
        pl.BlockSpec((TILE, M), lambda i: (i, 0)),
        pl.BlockSpec((TILE, IN), lambda i: (i, 0)),
        pl.BlockSpec((8, 128), lambda i: (0, 0)),
        pl.BlockSpec((8, 128), lambda i: (0, 0)),
    )
    bf = jnp.bfloat16
    scratch_shapes = [
        pltpu.VMEM((IN, D), bf), pltpu.VMEM((NRES, D, D), bf),
        pltpu.VMEM((D, M), bf), pltpu.VMEM((M, D), bf),
        pltpu.VMEM((NRES, D, D), bf), pltpu.VMEM((D, IN), bf),
        pltpu.VMEM((K, EDIM), bf), pltpu.VMEM((EDIM, K), bf),
        pltpu.VMEM((8, K), jnp.float32), pltpu.VMEM((8, 128), jnp.float32),
    ]
    y_hat, x_hat, loss_b, perp_b = pl.pallas_call(
        _fused_body,
        grid=(NTILES,),
        in_specs=in_specs,
        out_specs=out_specs,
        out_shape=out_shapes,
        scratch_shapes=scratch_shapes,
    )(*operands)

    return (loss_b[0, 0], x_hat, perp_b[0, 0], y_hat)
